# Initial kernel scaffold; baseline (speedup 1.0000x reference)
import jax
import jax.numpy as jnp
from jax.experimental import pallas as pl


def kernel(z, pos, batch, embedding, Wp, w_gate, W2, b2):
    raise NotImplementedError("write your pallas kernel here")


# speedup vs baseline: 10.4276x; 10.4276x over previous
import functools

import jax
import jax.numpy as jnp
from jax import lax
from jax.experimental import pallas as pl
from jax.experimental.pallas import tpu as pltpu
from jax.experimental.pallas import tpu_sc as plsc

N = 320000
NUM_SEG = 10000
D = 128
ZMAX = 100

NUM_TILES = 16
LANES = 128
B = 4096
N_PAD = 323584
K_WIN = N_PAD // (NUM_TILES * LANES)
NB = N_PAD // B


def _atom_scalar_body(z_ref, pos_ref, embT_ref, wpT_ref, wg_ref, w2_ref, y_ref):
    pos_b = pos_ref[...]
    z_b = z_ref[0]
    lift = lax.dot_general(wpT_ref[...], pos_b, (((1,), (0,)), ((), ())),
                           preferred_element_type=jnp.float32)
    lift = lift * jax.nn.sigmoid(lift)
    types = lax.broadcasted_iota(jnp.int32, (ZMAX, B), 0)
    oh = (types == z_b).astype(jnp.float32)
    eg = lax.dot_general(embT_ref[...], oh, (((1,), (0,)), ((), ())),
                         preferred_element_type=jnp.float32)
    u = eg + lift
    su = u * jax.nn.sigmoid(u)
    v = wg_ref[...] * w2_ref[...]
    y = jnp.sum(su * v, axis=0, keepdims=True)
    gidx = pl.program_id(0) * B + lax.broadcasted_iota(jnp.int32, (1, B), 1)
    y_ref[...] = jnp.where(gidx < N, y, 0.0)


def _atom_scalars(z3, posT, embT, wpT, wg, w2):
    return pl.pallas_call(
        _atom_scalar_body,
        grid=(NB,),
        in_specs=[
            pl.BlockSpec((1, 1, B), lambda i: (i, 0, 0)),
            pl.BlockSpec((3, B), lambda i: (0, i)),
            pl.BlockSpec((D, ZMAX), lambda i: (0, 0)),
            pl.BlockSpec((D, 3), lambda i: (0, 0)),
            pl.BlockSpec((D, 1), lambda i: (0, 0)),
            pl.BlockSpec((D, 1), lambda i: (0, 0)),
        ],
        out_specs=pl.BlockSpec((1, B), lambda i: (0, i)),
        out_shape=jax.ShapeDtypeStruct((1, N_PAD), jnp.float32),
    )(z3, posT, embT, wpT, wg, w2)


def _segsum_body(y_hbm, idx_hbm, init_hbm, out_hbm, yv, iv, acc):
    s = lax.axis_index("s")

    pltpu.sync_copy(y_hbm.at[s], yv)
    pltpu.sync_copy(idx_hbm.at[s], iv)

    @pl.when(s == 0)
    def _():
        pltpu.sync_copy(init_hbm, acc)

    plsc.subcore_barrier()

    def body(j, carry):
        pltpu.sync_copy(yv.at[j], acc.at[iv.at[j]], add=True)
        return carry

    lax.fori_loop(0, K_WIN, body, 0)

    plsc.subcore_barrier()

    @pl.when(s == 0)
    def _():
        pltpu.sync_copy(acc, out_hbm)


@functools.cache
def _build_segsum():
    return pl.kernel(
        _segsum_body,
        out_type=jax.ShapeDtypeStruct((NUM_SEG,), jnp.float32),
        mesh=plsc.VectorSubcoreMesh(core_axis_name="c", subcore_axis_name="s",
                                    num_cores=1, num_subcores=NUM_TILES),
        scratch_types=[
            pltpu.VMEM((K_WIN, LANES), jnp.float32),
            pltpu.VMEM((K_WIN, LANES), jnp.int32),
            pltpu.VMEM_SHARED((NUM_SEG,), jnp.float32),
        ],
    )


def kernel(z, pos, batch, embedding, Wp, w_gate, W2, b2):
    pad = N_PAD - N
    z3 = jnp.pad(z.astype(jnp.int32), (0, pad)).reshape(NB, 1, B)
    posT = jnp.pad(pos, ((0, pad), (0, 0))).T
    embT = embedding.T
    wpT = Wp.T
    wg = w_gate.reshape(D, 1)

    y = _atom_scalars(z3, posT, embT, wpT, wg, W2)

    y3 = y.reshape(NUM_TILES, K_WIN, LANES)
    idx3 = jnp.pad(batch.astype(jnp.int32), (0, pad)).reshape(
        NUM_TILES, K_WIN, LANES)
    init = jnp.broadcast_to(b2, (NUM_SEG,)).astype(jnp.float32)

    out = _build_segsum()(y3, idx3, init)
    return out.reshape(NUM_SEG, 1)
